# baseline (speedup 1.0000x reference)
import jax
import jax.numpy as jnp
import numpy as np
from jax.experimental import pallas as pl
from jax.experimental.pallas import tpu as pltpu

N = 1024
FEAT = 128
HID = 128
HEADS = 8
CH = HID // HEADS
NB = 512


def _expand_proj(a):
    tiled = jnp.concatenate([a] * HEADS, axis=1)
    lane = jax.lax.broadcasted_iota(jnp.int32, (HEADS, HID), 1)
    hrow = jax.lax.broadcasted_iota(jnp.int32, (HEADS, HID), 0)
    return jnp.where(lane // CH == hrow, tiled, 0.0)


def _gat_layer_kernel(inp_ref, inph_ref, adj_ref, W_ref, as_ref, ad_ref,
                      b_ref, out_ref):
    pid = pl.program_id(0)
    adj = adj_ref[...]
    row = jax.lax.broadcasted_iota(jnp.int32, (N, NB), 0)
    col = jax.lax.broadcasted_iota(jnp.int32, (N, NB), 1) + pid * NB
    mask_add = jnp.where(jnp.logical_or(row == col, adj != 0.0), 0.0, -1e4)
    ones_col = jnp.ones((N, 1), dtype=jnp.float32)

    h = jnp.dot(inp_ref[...], W_ref[...], preferred_element_type=jnp.float32)
    Bs = _expand_proj(as_ref[...])
    Bd = _expand_proj(ad_ref[...])
    al_s = jax.lax.dot_general(h, Bs, (((1,), (1,)), ((), ())),
                               preferred_element_type=jnp.float32)
    h_half = jnp.dot(inph_ref[...], W_ref[...],
                     preferred_element_type=jnp.float32)
    al_d_t = jax.lax.dot_general(Bd, h_half, (((1,), (1,)), ((), ())),
                                 preferred_element_type=jnp.float32)
    S = jnp.max(al_s, axis=0, keepdims=True)

    outs = []
    for hd in range(HEADS):
        s_col = al_s[:, hd:hd + 1]
        d_row = al_d_t[hd:hd + 1, :]
        Sh = S[:, hd:hd + 1]
        z = Sh + d_row
        mhat = jnp.maximum(z, 0.2 * z)
        w1 = s_col + (d_row - mhat)
        w2 = 0.2 * s_col + (0.2 * d_row - mhat)
        p = jnp.exp(jnp.maximum(w1, w2) + mask_add)
        h_aug = jnp.concatenate(
            [h[:, hd * CH:(hd + 1) * CH], ones_col], axis=1)
        o_aug = jax.lax.dot_general(p, h_aug, (((0,), (0,)), ((), ())),
                                    preferred_element_type=jnp.float32)
        outs.append(o_aug[:, :CH] / (o_aug[:, CH:CH + 1] + 1e-16))
    o = jnp.concatenate(outs, axis=1) + b_ref[...]
    out_ref[...] = jnp.where(o > 0.0, o, jnp.exp(jnp.minimum(o, 0.0)) - 1.0)


def _gat_layer(inp, adj, W, a_s, a_d, b):
    return pl.pallas_call(
        _gat_layer_kernel,
        grid=(N // NB,),
        in_specs=[
            pl.BlockSpec((N, FEAT), lambda j: (0, 0)),
            pl.BlockSpec((NB, FEAT), lambda j: (j, 0)),
            pl.BlockSpec((N, NB), lambda j: (0, j)),
            pl.BlockSpec((FEAT, HID), lambda j: (0, 0)),
            pl.BlockSpec((HEADS, CH), lambda j: (0, 0)),
            pl.BlockSpec((HEADS, CH), lambda j: (0, 0)),
            pl.BlockSpec((1, HID), lambda j: (0, 0)),
        ],
        out_specs=pl.BlockSpec((NB, HID), lambda j: (j, 0)),
        out_shape=jax.ShapeDtypeStruct((N, HID), jnp.float32),
        compiler_params=pltpu.CompilerParams(
            dimension_semantics=("parallel",)),
    )(inp, inp, adj, W, a_s, a_d, b)


@jax.jit
def kernel(x, adj, W1, a_src1, a_dst1, b1, W2, a_src2, a_dst2, b2):
    h1 = _gat_layer(x, adj, W1, a_src1, a_dst1, b1.reshape(1, HID))
    return _gat_layer(h1, adj, W2, a_src2, a_dst2, b2.reshape(1, HID))


# speedup vs baseline: 1.4570x; 1.4570x over previous
import jax
import jax.numpy as jnp
import numpy as np
from jax.experimental import pallas as pl
from jax.experimental.pallas import tpu as pltpu

N = 1024
FEAT = 128
HID = 128
HEADS = 8
CH = HID // HEADS


def _expand_proj(a):
    tiled = jnp.concatenate([a] * HEADS, axis=1)
    lane = jax.lax.broadcasted_iota(jnp.int32, (HEADS, HID), 1)
    hrow = jax.lax.broadcasted_iota(jnp.int32, (HEADS, HID), 0)
    return jnp.where(lane // CH == hrow, tiled, 0.0)


def _gat2_kernel(x_ref, adj_ref, W1_ref, as1_ref, ad1_ref, b1_ref,
                 W2_ref, as2_ref, ad2_ref, b2_ref, out_ref):
    adjt = jnp.transpose(adj_ref[...])
    row = jax.lax.broadcasted_iota(jnp.int32, (N, N), 0)
    col = jax.lax.broadcasted_iota(jnp.int32, (N, N), 1)
    mask_add = jnp.where(jnp.logical_or(row == col, adjt != 0.0), 0.0, -1e4)
    ones_col = jnp.ones((N, 1), dtype=jnp.float32)

    def layer(inp, W_ref, as_ref, ad_ref, b_ref):
        h = jnp.dot(inp, W_ref[...], preferred_element_type=jnp.float32)
        Bs = _expand_proj(as_ref[...])
        Bd = _expand_proj(ad_ref[...])
        al_s_t = jax.lax.dot_general(Bs, h, (((1,), (1,)), ((), ())),
                                     preferred_element_type=jnp.float32)
        al_d = jax.lax.dot_general(h, Bd, (((1,), (1,)), ((), ())),
                                   preferred_element_type=jnp.float32)
        S = jnp.max(al_s_t, axis=1, keepdims=True)
        outs = []
        for hd in range(HEADS):
            s_row = al_s_t[hd:hd + 1, :]
            d_col = al_d[:, hd:hd + 1]
            Sh = S[hd:hd + 1, :]
            z = Sh + d_col
            mhat = jnp.maximum(z, 0.2 * z)
            w1 = (d_col - mhat) + s_row
            w2 = (0.2 * d_col - mhat) + 0.2 * s_row
            p = jnp.exp(jnp.maximum(w1, w2) + mask_add)
            h_aug = jnp.concatenate(
                [h[:, hd * CH:(hd + 1) * CH], ones_col], axis=1)
            o_aug = jnp.dot(p, h_aug, preferred_element_type=jnp.float32)
            outs.append(o_aug[:, :CH] / (o_aug[:, CH:CH + 1] + 1e-16))
        return jnp.concatenate(outs, axis=1) + b_ref[...]

    h1 = layer(x_ref[...], W1_ref, as1_ref, ad1_ref, b1_ref)
    h1 = jnp.where(h1 > 0.0, h1, jnp.exp(jnp.minimum(h1, 0.0)) - 1.0)
    h2 = layer(h1, W2_ref, as2_ref, ad2_ref, b2_ref)
    out_ref[...] = jnp.where(h2 > 0.0, h2, jnp.exp(jnp.minimum(h2, 0.0)) - 1.0)


@jax.jit
def kernel(x, adj, W1, a_src1, a_dst1, b1, W2, a_src2, a_dst2, b2):
    return pl.pallas_call(
        _gat2_kernel,
        out_shape=jax.ShapeDtypeStruct((N, HID), jnp.float32),
    )(x, adj, W1, a_src1, a_dst1, b1.reshape(1, HID),
      W2, a_src2, a_dst2, b2.reshape(1, HID))


# speedup vs baseline: 1.4823x; 1.0174x over previous
import jax
import jax.numpy as jnp
import numpy as np
from jax.experimental import pallas as pl
from jax.experimental.pallas import tpu as pltpu

N = 1024
FEAT = 128
HID = 128
HEADS = 8
CH = HID // HEADS


def _expand_proj(a):
    tiled = jnp.concatenate([a] * HEADS, axis=1)
    lane = jax.lax.broadcasted_iota(jnp.int32, (HEADS, HID), 1)
    hrow = jax.lax.broadcasted_iota(jnp.int32, (HEADS, HID), 0)
    return jnp.where(lane // CH == hrow, tiled, 0.0)


def _gat2_kernel(x_ref, adj_ref, W1_ref, as1_ref, ad1_ref, b1_ref,
                 W2_ref, as2_ref, ad2_ref, b2_ref, out_ref):
    adjt = jnp.transpose(adj_ref[...])
    row = jax.lax.broadcasted_iota(jnp.int32, (N, N), 0)
    col = jax.lax.broadcasted_iota(jnp.int32, (N, N), 1)
    mask_add = jnp.where(jnp.logical_or(row == col, adjt != 0.0), 0.0, -1e4)
    ones_col = jnp.ones((N, 1), dtype=jnp.float32)

    def layer(inp, W_ref, as_ref, ad_ref, b_ref):
        h = jnp.dot(inp, W_ref[...], preferred_element_type=jnp.float32)
        Bs = _expand_proj(as_ref[...])
        Bd = _expand_proj(ad_ref[...])
        al_s_t = jax.lax.dot_general(Bs, h, (((1,), (1,)), ((), ())),
                                     preferred_element_type=jnp.float32)
        al_d = jax.lax.dot_general(h, Bd, (((1,), (1,)), ((), ())),
                                   preferred_element_type=jnp.float32)
        S = jnp.max(al_s_t, axis=1, keepdims=True)
        outs = []
        for hd in range(HEADS):
            s_row = al_s_t[hd:hd + 1, :]
            d_col = al_d[:, hd:hd + 1]
            Sh = S[hd:hd + 1, :]
            z = Sh + d_col
            mhat = jnp.maximum(z, 0.2 * z)
            w1 = (d_col - mhat) + s_row
            w2 = (0.2 * d_col - mhat) + 0.2 * s_row
            p = jnp.exp(jnp.maximum(w1, w2) + mask_add).astype(jnp.bfloat16)
            h_aug = jnp.concatenate(
                [h[:, hd * CH:(hd + 1) * CH], ones_col], axis=1)
            o_aug = jnp.dot(p, h_aug.astype(jnp.bfloat16),
                            preferred_element_type=jnp.float32)
            outs.append(o_aug[:, :CH] / (o_aug[:, CH:CH + 1] + 1e-16))
        return jnp.concatenate(outs, axis=1) + b_ref[...]

    h1 = layer(x_ref[...], W1_ref, as1_ref, ad1_ref, b1_ref)
    h1 = jnp.where(h1 > 0.0, h1, jnp.exp(jnp.minimum(h1, 0.0)) - 1.0)
    h2 = layer(h1, W2_ref, as2_ref, ad2_ref, b2_ref)
    out_ref[...] = jnp.where(h2 > 0.0, h2, jnp.exp(jnp.minimum(h2, 0.0)) - 1.0)


@jax.jit
def kernel(x, adj, W1, a_src1, a_dst1, b1, W2, a_src2, a_dst2, b2):
    return pl.pallas_call(
        _gat2_kernel,
        out_shape=jax.ShapeDtypeStruct((N, HID), jnp.float32),
    )(x, adj, W1, a_src1, a_dst1, b1.reshape(1, HID),
      W2, a_src2, a_dst2, b2.reshape(1, HID))


# speedup vs baseline: 1.6097x; 1.0859x over previous
import jax
import jax.numpy as jnp
import numpy as np
from jax.experimental import pallas as pl
from jax.experimental.pallas import tpu as pltpu

N = 1024
FEAT = 128
HID = 128
HEADS = 8
CH = HID // HEADS


def _expand_proj(a):
    tiled = jnp.concatenate([a] * HEADS, axis=1)
    lane = jax.lax.broadcasted_iota(jnp.int32, (HEADS, HID), 1)
    hrow = jax.lax.broadcasted_iota(jnp.int32, (HEADS, HID), 0)
    return jnp.where(lane // CH == hrow, tiled, 0.0)


def _gat2_kernel(x_ref, adj_ref, W1_ref, as1_ref, ad1_ref, b1_ref,
                 W2_ref, as2_ref, ad2_ref, b2_ref, out_ref):
    adjt = jnp.transpose(adj_ref[...])
    row = jax.lax.broadcasted_iota(jnp.int32, (N, N), 0)
    col = jax.lax.broadcasted_iota(jnp.int32, (N, N), 1)
    maskf = jnp.where(jnp.logical_or(row == col, adjt != 0.0),
                      1.0, 0.0).astype(jnp.bfloat16)
    ones_col = jnp.ones((N, 1), dtype=jnp.float32)
    LOG2E = 1.4426950408889634

    def layer(inp, W_ref, as_ref, ad_ref, b_ref):
        h = jnp.dot(inp, W_ref[...], preferred_element_type=jnp.float32)
        Bs = _expand_proj(as_ref[...])
        Bd = _expand_proj(ad_ref[...])
        al_s_t = jax.lax.dot_general(Bs, h, (((1,), (1,)), ((), ())),
                                     preferred_element_type=jnp.float32)
        al_d = jax.lax.dot_general(h, Bd, (((1,), (1,)), ((), ())),
                                   preferred_element_type=jnp.float32)
        S = jnp.max(al_s_t, axis=1, keepdims=True)
        outs = []
        for hd in range(HEADS):
            s_row = al_s_t[hd:hd + 1, :]
            d_col = al_d[:, hd:hd + 1]
            Sh = S[hd:hd + 1, :]
            z = Sh + d_col
            mhat = jnp.maximum(z, 0.2 * z)
            w1 = LOG2E * (d_col - mhat) + LOG2E * s_row
            w2 = (LOG2E * 0.2) * (d_col - 5.0 * mhat) \
                + (LOG2E * 0.2) * s_row
            p = (jnp.exp2(jnp.maximum(w1, w2)).astype(jnp.bfloat16)
                 * maskf)
            h_aug = jnp.concatenate(
                [h[:, hd * CH:(hd + 1) * CH], ones_col], axis=1)
            o_aug = jnp.dot(p, h_aug.astype(jnp.bfloat16),
                            preferred_element_type=jnp.float32)
            outs.append(o_aug[:, :CH] / (o_aug[:, CH:CH + 1] + 1e-16))
        return jnp.concatenate(outs, axis=1) + b_ref[...]

    h1 = layer(x_ref[...], W1_ref, as1_ref, ad1_ref, b1_ref)
    h1 = jnp.where(h1 > 0.0, h1, jnp.exp(jnp.minimum(h1, 0.0)) - 1.0)
    h2 = layer(h1, W2_ref, as2_ref, ad2_ref, b2_ref)
    out_ref[...] = jnp.where(h2 > 0.0, h2, jnp.exp(jnp.minimum(h2, 0.0)) - 1.0)


@jax.jit
def kernel(x, adj, W1, a_src1, a_dst1, b1, W2, a_src2, a_dst2, b2):
    return pl.pallas_call(
        _gat2_kernel,
        out_shape=jax.ShapeDtypeStruct((N, HID), jnp.float32),
    )(x, adj, W1, a_src1, a_dst1, b1.reshape(1, HID),
      W2, a_src2, a_dst2, b2.reshape(1, HID))


# speedup vs baseline: 2.3376x; 1.4522x over previous
import jax
import jax.numpy as jnp
import numpy as np
from jax.experimental import pallas as pl
from jax.experimental.pallas import tpu as pltpu

N = 1024
FEAT = 128
HID = 128
HEADS = 8
CH = HID // HEADS


def _expand_proj(a):
    tiled = jnp.concatenate([a] * HEADS, axis=1)
    lane = jax.lax.broadcasted_iota(jnp.int32, (HEADS, HID), 1)
    hrow = jax.lax.broadcasted_iota(jnp.int32, (HEADS, HID), 0)
    return jnp.where(lane // CH == hrow, tiled, 0.0)


def _gat2_kernel(x_ref, adj_ref, W1_ref, as1_ref, ad1_ref, b1_ref,
                 W2_ref, as2_ref, ad2_ref, b2_ref, out_ref):
    adj = adj_ref[...]
    row = jax.lax.broadcasted_iota(jnp.int32, (N, N), 0)
    col = jax.lax.broadcasted_iota(jnp.int32, (N, N), 1)
    maskf = jnp.where(jnp.logical_or(row == col, adj != 0.0),
                      1.0, 0.0).astype(jnp.bfloat16)
    ones_row = jnp.ones((1, N), dtype=jnp.float32)
    LOG2E = 1.4426950408889634

    x_t = jnp.transpose(x_ref[...])

    def layer(inp_t, W_ref, as_ref, ad_ref, b_ref):
        h_t = jnp.dot(jnp.transpose(W_ref[...]), inp_t,
                      preferred_element_type=jnp.float32)
        Bs = _expand_proj(as_ref[...])
        Bd = _expand_proj(ad_ref[...])
        al_d_t = jnp.dot(Bd, h_t, preferred_element_type=jnp.float32)
        al_s = jax.lax.dot_general(h_t, Bs, (((0,), (1,)), ((), ())),
                                   preferred_element_type=jnp.float32)
        S = jnp.max(al_s, axis=0, keepdims=True)
        b_col = jnp.transpose(b_ref[...])
        outs = []
        for hd in range(HEADS):
            s_col = al_s[:, hd:hd + 1]
            d_row = al_d_t[hd:hd + 1, :]
            Sh = S[:, hd:hd + 1]
            z = Sh + d_row
            mhat = jnp.maximum(z, 0.2 * z)
            w1 = LOG2E * s_col + LOG2E * (d_row - mhat)
            w2 = (LOG2E * 0.2) * s_col \
                + (LOG2E * 0.2) * (d_row - 5.0 * mhat)
            p = (jnp.exp2(jnp.maximum(w1, w2)).astype(jnp.bfloat16)
                 * maskf)
            h_aug_t = jnp.concatenate(
                [h_t[hd * CH:(hd + 1) * CH, :], ones_row], axis=0)
            o_aug_t = jnp.dot(h_aug_t.astype(jnp.bfloat16), p,
                              preferred_element_type=jnp.float32)
            outs.append(o_aug_t[:CH, :]
                        / (o_aug_t[CH:CH + 1, :] + 1e-16))
        return jnp.concatenate(outs, axis=0) + b_col

    h1_t = layer(x_t, W1_ref, as1_ref, ad1_ref, b1_ref)
    h1_t = jnp.where(h1_t > 0.0, h1_t,
                     jnp.exp(jnp.minimum(h1_t, 0.0)) - 1.0)
    h2_t = layer(h1_t, W2_ref, as2_ref, ad2_ref, b2_ref)
    h2_t = jnp.where(h2_t > 0.0, h2_t,
                     jnp.exp(jnp.minimum(h2_t, 0.0)) - 1.0)
    out_ref[...] = jnp.transpose(h2_t)


@jax.jit
def kernel(x, adj, W1, a_src1, a_dst1, b1, W2, a_src2, a_dst2, b2):
    return pl.pallas_call(
        _gat2_kernel,
        out_shape=jax.ShapeDtypeStruct((N, HID), jnp.float32),
    )(x, adj, W1, a_src1, a_dst1, b1.reshape(1, HID),
      W2, a_src2, a_dst2, b2.reshape(1, HID))
